# baseline (speedup 1.0000x reference)
import functools

import jax
import jax.numpy as jnp
from jax import lax
from jax.experimental import pallas as pl
from jax.experimental.pallas import tpu as pltpu
from jax.experimental.pallas import tpu_sc as plsc

N = 10000
E = 320000
C = 128
CW = C // 2
EPS = 1e-5

NC = 2
NS = 16
NW = NC * NS
EPW = E // NW
CH = 80
NCH = EPW // CH
NL = 16
WG = CW // NL
NP = 10240
NPS = NP // NS
HR = NP // 8
HT = HR // NS
CH3 = 40
NCH3 = EPW // CH3


@functools.cache
def _mesh():
    return plsc.VectorSubcoreMesh(
        core_axis_name="c", subcore_axis_name="s", num_cores=NC, num_subcores=NS
    )


_HI_MASK = -65536


def _pack_words(lo, hi):
    li = lax.bitcast_convert_type(lo, jnp.int32)
    hj = lax.bitcast_convert_type(hi, jnp.int32)
    lr = li + 0x7FFF + ((li >> 16) & 1)
    hr = hj + 0x7FFF + ((hj >> 16) & 1)
    return ((lr >> 16) & 0xFFFF) | (hr & _HI_MASK)


def _unpack_words(wi):
    lo = lax.bitcast_convert_type(wi << 16, jnp.float32)
    hi = lax.bitcast_convert_type(wi & _HI_MASK, jnp.float32)
    return lo, hi


def _m3_matrix(transpose=False):
    a = lax.broadcasted_iota(jnp.int32, (C, C), 1 if transpose else 0)
    b = lax.broadcasted_iota(jnp.int32, (C, C), 0 if transpose else 1)
    q = a % CW
    chan3 = 32 * (q // NL) + (q % NL) + NL * (a // CW)
    return jnp.where(b == chan3, 1.0, 0.0)


def _p0_body(x_ref, w1_ref, u_ref, v_ref):
    xb = x_ref[...]
    w1a = w1_ref[:C, :]
    w1b = w1_ref[C:, :]
    u_ref[...] = jnp.dot(xb, w1a - w1b, preferred_element_type=jnp.float32)
    v_ref[...] = jnp.dot(xb, w1b, preferred_element_type=jnp.float32)


_P0_RB = 1000


def _p0(x, W1):
    return pl.pallas_call(
        _p0_body,
        grid=(N // _P0_RB,),
        in_specs=[
            pl.BlockSpec((_P0_RB, C), lambda i: (i, 0)),
            pl.BlockSpec((2 * C, C), lambda i: (0, 0)),
        ],
        out_specs=[pl.BlockSpec((_P0_RB, C), lambda i: (i, 0))] * 2,
        out_shape=[jax.ShapeDtypeStruct((N, C), jnp.float32)] * 2,
    )(x, W1)


def _p1_real_body(u_hbm, v_hbm, dst_hbm, src_hbm, h1_hbm, s1_hbm, q1_hbm,
                  cnt_hbm,
                  idxd, idxs, idxq, ubuf, vbuf, obuf, pat, stbuf, cnt_sh,
                  sem_u, sem_v, sem_o, sem_p, sem_i):
    cid = lax.axis_index("c")
    sid = lax.axis_index("s")
    wid = sid * NC + cid
    base = wid * EPW
    zero_v = jnp.zeros((NL,), jnp.float32)
    lane = lax.iota(jnp.int32, NL)

    cpi = pltpu.async_copy(dst_hbm.at[wid], idxd, sem_i)
    cpj = pltpu.async_copy(src_hbm.at[wid], idxs, sem_i)

    def zrow_body(r, _):
        for g in range(C // NL):
            pat[0, r, pl.ds(g * NL, NL)] = zero_v
        return 0

    lax.fori_loop(0, CH, zrow_body, 0)
    cpi.wait()
    cpj.wait()

    pltpu.sync_copy(pat.at[0], cnt_sh.at[pl.ds(sid * HT, HT)])

    plsc.subcore_barrier()

    pltpu.async_copy(u_hbm.at[idxd.at[0]], ubuf.at[0], sem_u)
    pltpu.async_copy(v_hbm.at[idxs.at[0]], vbuf.at[0], sem_v)

    def do_chunk(j, b, acc):
        pltpu.make_async_copy(u_hbm.at[idxd.at[j]], ubuf.at[b], sem_u).wait()
        pltpu.make_async_copy(v_hbm.at[idxs.at[j]], vbuf.at[b], sem_v).wait()

        @pl.when(j + 1 < NCH)
        def _():
            pltpu.async_copy(u_hbm.at[idxd.at[j + 1]], ubuf.at[1 - b], sem_u)
            pltpu.async_copy(v_hbm.at[idxs.at[j + 1]], vbuf.at[1 - b], sem_v)

        @pl.when(j >= 2)
        def _():
            pltpu.make_async_copy(
                obuf.at[b], h1_hbm.at[pl.ds(base + (j - 2) * CH, CH)], sem_o
            ).wait()
            pltpu.make_async_copy(
                pat.at[b], cnt_sh.at[idxq.at[b]], sem_p
            ).wait()

        for g in range(CH // NL):
            sl = pl.ds(g * NL, NL)
            idxq[b, sl] = lax.shift_right_logical(idxd[j, sl], 3)

        def egroup(gi, a):
            e0 = gi * NL
            d16 = idxd[j, pl.ds(e0, NL)]
            dcol16 = (d16 & 7) * NL
            a = list(a)
            for l in range(NL):
                e = e0 + l
                dcol = dcol16[l]
                for g in range(C // NL):
                    pat[b, e, pl.ds(g * NL, NL)] = jnp.where(
                        lane + g * NL == dcol, 1.0, 0.0
                    )
                hs = []
                for g in range(C // NL):
                    sl = pl.ds(g * NL, NL)
                    h = ubuf[b, e, sl] + vbuf[b, e, sl]
                    hs.append(h)
                    a[g] = a[g] + h
                    a[C // NL + g] = a[C // NL + g] + h * h
                for k in range(WG):
                    lo = lax.bitcast_convert_type(hs[2 * k], jnp.int32)
                    hi = lax.bitcast_convert_type(hs[2 * k + 1], jnp.int32)
                    obuf[b, e, pl.ds(k * NL, NL)] = (
                        ((lo + 0x8000) >> 16) & 0xFFFF
                    ) | ((hi + 0x8000) & _HI_MASK)
            return tuple(a)

        acc = lax.fori_loop(0, CH // NL, egroup, acc)
        pltpu.async_copy(obuf.at[b], h1_hbm.at[pl.ds(base + j * CH, CH)], sem_o)
        pltpu.async_copy(pat.at[b], cnt_sh.at[idxq.at[b]], sem_p, add=True)
        return acc

    zacc = tuple(jnp.zeros((NL,), jnp.float32) for _ in range(2 * (C // NL)))

    def outer(io, acc):
        acc = do_chunk(io * 2, 0, acc)
        acc = do_chunk(io * 2 + 1, 1, acc)
        return acc

    acc = lax.fori_loop(0, NCH // 2, outer, zacc)
    acc = do_chunk(NCH - 1, 0, acc)

    pltpu.make_async_copy(obuf.at[0], h1_hbm.at[pl.ds(base, CH)], sem_o).wait()
    pltpu.make_async_copy(obuf.at[1], h1_hbm.at[pl.ds(base, CH)], sem_o).wait()
    pltpu.make_async_copy(pat.at[0], cnt_sh.at[idxq.at[0]], sem_p).wait()
    pltpu.make_async_copy(pat.at[1], cnt_sh.at[idxq.at[1]], sem_p).wait()

    plsc.subcore_barrier()

    pltpu.sync_copy(
        cnt_sh.at[pl.ds(sid * HT, HT)],
        cnt_hbm.at[pl.ds(cid * HR + sid * HT, HT)],
    )

    for g in range(C // NL):
        stbuf[0, 0, pl.ds(g * NL, NL)] = acc[g]
        stbuf[1, 0, pl.ds(g * NL, NL)] = acc[C // NL + g]
    pltpu.sync_copy(stbuf.at[0], s1_hbm.at[wid])
    pltpu.sync_copy(stbuf.at[1], q1_hbm.at[wid])


@functools.cache
def _p1():
    return pl.kernel(
        _p1_real_body,
        out_type=(
            jax.ShapeDtypeStruct((E, CW), jnp.int32),
            jax.ShapeDtypeStruct((NW, 1, C), jnp.float32),
            jax.ShapeDtypeStruct((NW, 1, C), jnp.float32),
            jax.ShapeDtypeStruct((NC * HR, C), jnp.float32),
        ),
        mesh=_mesh(),
        scratch_types=[
            pltpu.VMEM((NCH, CH), jnp.int32),
            pltpu.VMEM((NCH, CH), jnp.int32),
            pltpu.VMEM((2, CH), jnp.int32),
            pltpu.VMEM((2, CH, C), jnp.float32),
            pltpu.VMEM((2, CH, C), jnp.float32),
            pltpu.VMEM((2, CH, CW), jnp.int32),
            pltpu.VMEM((2, CH, C), jnp.float32),
            pltpu.VMEM((2, 1, C), jnp.float32),
            pltpu.VMEM_SHARED((HR, C), jnp.float32),
            pltpu.SemaphoreType.DMA,
            pltpu.SemaphoreType.DMA,
            pltpu.SemaphoreType.DMA,
            pltpu.SemaphoreType.DMA,
            pltpu.SemaphoreType.DMA,
        ],
    )


_P2_RB = 2000
_P2_NSTEP = E // _P2_RB


def _p2_body(s1_ref, q1_ref, g1_ref, be1_ref, w2_ref, g2_ref, be2_ref,
             h1_ref, h2_ref, ab2_ref, acc_ref, w2s_ref, bn1_ref):
    i = pl.program_id(0)

    @pl.when(i == 0)
    def _():
        m3 = _m3_matrix()
        m3t = _m3_matrix(transpose=True)
        s1 = jnp.sum(s1_ref[...], axis=0)
        q1 = jnp.sum(q1_ref[...], axis=0)
        mn1 = s1 / E
        var1 = q1 / E - mn1 * mn1
        inv1 = lax.rsqrt(var1 + EPS)
        a1 = g1_ref[0, :] * inv1
        c1 = be1_ref[0, :] - mn1 * a1
        bn1_ref[0, :] = jnp.dot(m3, a1, preferred_element_type=jnp.float32)
        bn1_ref[1, :] = jnp.dot(m3, c1, preferred_element_type=jnp.float32)
        w2s_ref[...] = jnp.dot(
            m3, jnp.dot(w2_ref[...], m3t, preferred_element_type=jnp.float32),
            preferred_element_type=jnp.float32)
        acc_ref[...] = jnp.zeros_like(acc_ref)
        ab2_ref[...] = jnp.zeros_like(ab2_ref)

    a1s = bn1_ref[0, :]
    c1s = bn1_ref[1, :]
    ev, od = _unpack_words(h1_ref[...])
    h1s = jnp.concatenate([ev, od], axis=1)
    z = jnp.maximum(h1s * a1s[None, :] + c1s[None, :], 0.0)
    h2s = jnp.dot(z.astype(jnp.bfloat16),
                  w2s_ref[...].astype(jnp.bfloat16),
                  preferred_element_type=jnp.float32)
    h2_ref[...] = _pack_words(h2s[:, :CW], h2s[:, CW:])
    acc_ref[0, :] += jnp.sum(h2s, axis=0)
    acc_ref[1, :] += jnp.sum(h2s * h2s, axis=0)

    @pl.when(i == _P2_NSTEP - 1)
    def _():
        m3 = _m3_matrix()
        mn2 = jnp.dot(acc_ref[0, :] / E, m3,
                      preferred_element_type=jnp.float32)
        q2 = jnp.dot(acc_ref[1, :] / E, m3,
                     preferred_element_type=jnp.float32)
        var2 = q2 - mn2 * mn2
        inv2 = lax.rsqrt(var2 + EPS)
        a2 = g2_ref[0, :] * inv2
        c2 = be2_ref[0, :] - mn2 * a2
        ab2_ref[0, :] = jnp.dot(m3, a2, preferred_element_type=jnp.float32)
        ab2_ref[1, :] = jnp.dot(m3, c2, preferred_element_type=jnp.float32)


def _p2(s1, q1, g1, be1, W2, g2, be2, h1):
    return pl.pallas_call(
        _p2_body,
        grid=(_P2_NSTEP,),
        in_specs=[
            pl.BlockSpec((NW, C), lambda i: (0, 0)),
            pl.BlockSpec((NW, C), lambda i: (0, 0)),
            pl.BlockSpec((1, C), lambda i: (0, 0)),
            pl.BlockSpec((1, C), lambda i: (0, 0)),
            pl.BlockSpec((C, C), lambda i: (0, 0)),
            pl.BlockSpec((1, C), lambda i: (0, 0)),
            pl.BlockSpec((1, C), lambda i: (0, 0)),
            pl.BlockSpec((_P2_RB, CW), lambda i: (i, 0)),
        ],
        out_specs=[
            pl.BlockSpec((_P2_RB, CW), lambda i: (i, 0)),
            pl.BlockSpec((8, C), lambda i: (0, 0)),
        ],
        out_shape=[
            jax.ShapeDtypeStruct((E, CW), jnp.int32),
            jax.ShapeDtypeStruct((8, C), jnp.float32),
        ],
        scratch_shapes=[
            pltpu.VMEM((8, C), jnp.float32),
            pltpu.VMEM((C, C), jnp.float32),
            pltpu.VMEM((8, C), jnp.float32),
        ],
    )(s1, q1, g1, be1, W2, g2, be2, h1)


def _p3_body(h2_hbm, dst_hbm, ab2_hbm, part_hbm,
             idxc, hbuf, sbuf, abuf, acc_sh,
             sem_h, sem_s, sem_i, sem_z):
    cid = lax.axis_index("c")
    sid = lax.axis_index("s")
    wid = sid * NC + cid
    base = wid * EPW
    row0 = sid * NPS
    zero_v = jnp.zeros((NL,), jnp.float32)

    cpa = pltpu.async_copy(ab2_hbm.at[pl.ds(0, 2)], abuf, sem_z)

    def fill_zrow(r, _):
        for g in range(C // NL):
            sbuf[0, r, pl.ds(g * NL, NL)] = zero_v
        return 0

    lax.fori_loop(0, CH, fill_zrow, 0)
    cpa.wait()

    nfull = NPS // CH
    for k in range(nfull):
        pltpu.async_copy(sbuf.at[0], acc_sh.at[pl.ds(row0 + k * CH, CH)], sem_z)
    for k in range(nfull):
        pltpu.make_async_copy(
            sbuf.at[0], acc_sh.at[pl.ds(row0, CH)], sem_z
        ).wait()

    plsc.subcore_barrier()

    ax = [abuf[0, pl.ds(k * NL, NL)] for k in range(WG)]
    ay = [abuf[0, pl.ds(CW + k * NL, NL)] for k in range(WG)]
    cx = [abuf[1, pl.ds(k * NL, NL)] for k in range(WG)]
    cy = [abuf[1, pl.ds(CW + k * NL, NL)] for k in range(WG)]

    pltpu.async_copy(dst_hbm.at[wid, pl.ds(0, 1)], idxc.at[0], sem_i)
    pltpu.async_copy(dst_hbm.at[wid, pl.ds(1, 1)], idxc.at[1], sem_i)
    pltpu.async_copy(h2_hbm.at[pl.ds(base, CH)], hbuf.at[0], sem_h)

    def do_chunk(j, b):
        jm4 = lax.rem(j, 4)
        pltpu.make_async_copy(
            h2_hbm.at[pl.ds(base + j * CH, CH)], hbuf.at[b], sem_h
        ).wait()
        pltpu.make_async_copy(dst_hbm.at[wid, pl.ds(j, 1)], idxc.at[jm4], sem_i).wait()

        @pl.when(j + 1 < NCH)
        def _():
            pltpu.async_copy(
                h2_hbm.at[pl.ds(base + (j + 1) * CH, CH)], hbuf.at[1 - b],
                sem_h,
            )

        @pl.when(j >= 2)
        def _():
            pltpu.make_async_copy(
                sbuf.at[b], acc_sh.at[idxc.at[jm4, 0]], sem_s
            ).wait()

        @pl.when(j + 2 < NCH)
        def _():
            pltpu.async_copy(
                dst_hbm.at[wid, pl.ds(j + 2, 1)], idxc.at[lax.rem(j + 2, 4)],
                sem_i,
            )

        def edge(e, _c):
            for k in range(WG):
                wi = hbuf[b, e, pl.ds(k * NL, NL)]
                hx = lax.bitcast_convert_type(wi << 16, jnp.float32)
                hy = lax.bitcast_convert_type(wi & _HI_MASK, jnp.float32)
                sbuf[b, e, pl.ds(k * NL, NL)] = jnp.maximum(
                    hx * ax[k] + cx[k], 0.0
                )
                sbuf[b, e, pl.ds(CW + k * NL, NL)] = jnp.maximum(
                    hy * ay[k] + cy[k], 0.0
                )
            return 0

        lax.fori_loop(0, CH, edge, 0)
        pltpu.async_copy(sbuf.at[b], acc_sh.at[idxc.at[jm4, 0]], sem_s, add=True)

    def outer(io, carry):
        do_chunk(io * 2, 0)
        do_chunk(io * 2 + 1, 1)
        return carry

    lax.fori_loop(0, NCH // 2, outer, 0)
    do_chunk(NCH - 1, 0)

    pltpu.make_async_copy(sbuf.at[0], acc_sh.at[idxc.at[0, 0]], sem_s).wait()
    pltpu.make_async_copy(sbuf.at[1], acc_sh.at[idxc.at[1, 0]], sem_s).wait()

    plsc.subcore_barrier()

    cpo = pltpu.async_copy(
        acc_sh.at[pl.ds(row0, NPS)], part_hbm.at[pl.ds(cid * NP + row0, NPS)],
        sem_z,
    )
    cpo.wait()


@functools.cache
def _p3():
    return pl.kernel(
        _p3_body,
        out_type=jax.ShapeDtypeStruct((NC * NP, C), jnp.float32),
        mesh=_mesh(),
        scratch_types=[
            pltpu.VMEM((4, 1, CH), jnp.int32),
            pltpu.VMEM((2, CH, CW), jnp.int32),
            pltpu.VMEM((2, CH, C), jnp.float32),
            pltpu.VMEM((2, C), jnp.float32),
            pltpu.VMEM_SHARED((NP, C), jnp.float32),
            pltpu.SemaphoreType.DMA,
            pltpu.SemaphoreType.DMA,
            pltpu.SemaphoreType.DMA,
            pltpu.SemaphoreType.DMA,
        ],
    )


_P4_RB = 1280
_P4_HR = _P4_RB // 8


def _p4_body(p0_ref, p1_ref, h0_ref, h1_ref, out_ref):
    hs = h0_ref[0] + h1_ref[0]
    colh = lax.broadcasted_iota(jnp.int32, (C, 8), 0) // NL
    sh = lax.broadcasted_iota(jnp.int32, (C, 8), 1)
    sel = jnp.where(colh == sh, 1.0, 0.0)
    cnt = jnp.dot(hs, sel, preferred_element_type=jnp.float32)
    den = jnp.clip(cnt, 1.0, None)
    p = p0_ref[0] + p1_ref[0]
    p = (p.reshape(_P4_HR, 8, C) / den[:, :, None]).reshape(_P4_RB, C)
    out_ref[...] = jnp.dot(p, _m3_matrix(),
                           preferred_element_type=jnp.float32)


def _p4(part, hist):
    nb = NP // _P4_RB
    part = part.reshape(NC, NP, C)
    hist = hist.reshape(NC, HR, C)
    return pl.pallas_call(
        _p4_body,
        grid=(nb,),
        in_specs=[
            pl.BlockSpec((1, _P4_RB, C), lambda i: (0, i, 0)),
            pl.BlockSpec((1, _P4_RB, C), lambda i: (1, i, 0)),
            pl.BlockSpec((1, _P4_HR, C), lambda i: (0, i, 0)),
            pl.BlockSpec((1, _P4_HR, C), lambda i: (1, i, 0)),
        ],
        out_specs=pl.BlockSpec((_P4_RB, C), lambda i: (i, 0)),
        out_shape=jax.ShapeDtypeStruct((NP, C), jnp.float32),
    )(part, part, hist, hist)


@jax.jit
def kernel(x, edge_index, W1, b1, g1, be1, W2, b2, g2, be2):
    del b1, b2
    src = edge_index[0].reshape(NW, NCH, CH)
    dst = edge_index[1].reshape(NW, NCH, CH)
    u, v = _p0(x, W1)
    h1, s1, q1, hist = _p1()(u, v, dst, src)
    s1 = s1.reshape(NW, C)
    q1 = q1.reshape(NW, C)
    h2, ab2 = _p2(s1, q1, g1.reshape(1, C), be1.reshape(1, C), W2,
                  g2.reshape(1, C), be2.reshape(1, C), h1)
    part = _p3()(h2, dst, ab2)
    out = _p4(part, hist)
    return out[:N]


# speedup vs baseline: 1.0242x; 1.0242x over previous
import functools

import jax
import jax.numpy as jnp
from jax import lax
from jax.experimental import pallas as pl
from jax.experimental.pallas import tpu as pltpu
from jax.experimental.pallas import tpu_sc as plsc

N = 10000
E = 320000
C = 128
EPS = 1e-5

NC = 2
NS = 16
NW = NC * NS
EPW = E // NW
CH = 80
NCH = EPW // CH
NL = 16
CG = C // NL
NP = 10240
NPS = NP // NS
NHR = NP // 8
NHT = NHR // NS


@functools.cache
def _mesh():
    return plsc.VectorSubcoreMesh(
        core_axis_name="c", subcore_axis_name="s", num_cores=NC, num_subcores=NS
    )


def _p0_body(x_ref, w1_ref, u_ref, v_ref):
    xb = x_ref[...]
    w1a = w1_ref[:C, :]
    w1b = w1_ref[C:, :]
    u_ref[...] = jnp.dot(xb, w1a - w1b, preferred_element_type=jnp.float32)
    v_ref[...] = jnp.dot(xb, w1b, preferred_element_type=jnp.float32)


_P0_RB = 1000


def _p0(x, W1):
    return pl.pallas_call(
        _p0_body,
        grid=(N // _P0_RB,),
        in_specs=[
            pl.BlockSpec((_P0_RB, C), lambda i: (i, 0)),
            pl.BlockSpec((2 * C, C), lambda i: (0, 0)),
        ],
        out_specs=[pl.BlockSpec((_P0_RB, C), lambda i: (i, 0))] * 2,
        out_shape=[jax.ShapeDtypeStruct((N, C), jnp.float32)] * 2,
    )(x, W1)


def _p1_real_body(u_hbm, v_hbm, dst_hbm, src_hbm, h1_hbm, s1_hbm, q1_hbm,
                  cnt_hbm,
                  idxd, idxs, idxq, ubuf, vbuf, obuf, pat, stbuf, cnt_sh,
                  sem_u, sem_v, sem_o, sem_p, sem_i):
    cid = lax.axis_index("c")
    sid = lax.axis_index("s")
    wid = sid * NC + cid
    base = wid * EPW
    lane = lax.iota(jnp.int32, NL)
    zero_v = jnp.zeros((NL,), jnp.float32)

    cpi = pltpu.async_copy(dst_hbm.at[wid], idxd, sem_i)
    cpj = pltpu.async_copy(src_hbm.at[wid], idxs, sem_i)

    def zrow_body(r, _):
        for g in range(CG):
            pat[0, r, pl.ds(g * NL, NL)] = zero_v
        return 0

    lax.fori_loop(0, CH, zrow_body, 0)
    cpi.wait()
    cpj.wait()

    pltpu.sync_copy(pat.at[0], cnt_sh.at[pl.ds(sid * NHT, NHT)])

    plsc.subcore_barrier()

    pltpu.async_copy(u_hbm.at[idxd.at[0]], ubuf.at[0], sem_u)
    pltpu.async_copy(v_hbm.at[idxs.at[0]], vbuf.at[0], sem_v)

    def do_chunk(j, b, acc):
        pltpu.make_async_copy(u_hbm.at[idxd.at[j]], ubuf.at[b], sem_u).wait()
        pltpu.make_async_copy(v_hbm.at[idxs.at[j]], vbuf.at[b], sem_v).wait()

        @pl.when(j + 1 < NCH)
        def _():
            pltpu.async_copy(u_hbm.at[idxd.at[j + 1]], ubuf.at[1 - b], sem_u)
            pltpu.async_copy(v_hbm.at[idxs.at[j + 1]], vbuf.at[1 - b], sem_v)

        @pl.when(j >= 2)
        def _():
            pltpu.make_async_copy(
                obuf.at[b], h1_hbm.at[pl.ds(base + (j - 2) * CH, CH)], sem_o
            ).wait()
            pltpu.make_async_copy(
                pat.at[b], cnt_sh.at[idxq.at[b]], sem_p
            ).wait()

        for g in range(CH // NL):
            sl = pl.ds(g * NL, NL)
            idxq[b, sl] = lax.shift_right_logical(idxd[j, sl], 3)

        def egroup(gi, a):
            e0 = gi * NL
            d16 = idxd[j, pl.ds(e0, NL)]
            dcol16 = (d16 & 7) * NL
            a = list(a)
            for l in range(NL):
                e = e0 + l
                dcol = dcol16[l]
                for g in range(CG):
                    sl = pl.ds(g * NL, NL)
                    h = ubuf[b, e, sl] + vbuf[b, e, sl]
                    obuf[b, e, sl] = h
                    a[g] = a[g] + h
                    a[CG + g] = a[CG + g] + h * h
                    pat[b, e, sl] = jnp.where(lane + g * NL == dcol, 1.0, 0.0)
            return tuple(a)

        acc = lax.fori_loop(0, CH // NL, egroup, acc)
        pltpu.async_copy(obuf.at[b], h1_hbm.at[pl.ds(base + j * CH, CH)], sem_o)
        pltpu.async_copy(pat.at[b], cnt_sh.at[idxq.at[b]], sem_p, add=True)
        return acc

    zacc = tuple(zero_v for _ in range(2 * CG))

    def outer(io, acc):
        acc = do_chunk(io * 2, 0, acc)
        acc = do_chunk(io * 2 + 1, 1, acc)
        return acc

    acc = lax.fori_loop(0, NCH // 2, outer, zacc)
    acc = do_chunk(NCH - 1, 0, acc)

    pltpu.make_async_copy(obuf.at[0], h1_hbm.at[pl.ds(base, CH)], sem_o).wait()
    pltpu.make_async_copy(obuf.at[1], h1_hbm.at[pl.ds(base, CH)], sem_o).wait()
    pltpu.make_async_copy(pat.at[0], cnt_sh.at[idxq.at[0]], sem_p).wait()
    pltpu.make_async_copy(pat.at[1], cnt_sh.at[idxq.at[1]], sem_p).wait()

    plsc.subcore_barrier()

    pltpu.sync_copy(
        cnt_sh.at[pl.ds(sid * NHT, NHT)],
        cnt_hbm.at[pl.ds(cid * NHR + sid * NHT, NHT)],
    )

    for g in range(CG):
        sl = pl.ds(g * NL, NL)
        stbuf[0, 0, sl] = acc[g]
        stbuf[1, 0, sl] = acc[CG + g]
    pltpu.sync_copy(stbuf.at[0], s1_hbm.at[wid])
    pltpu.sync_copy(stbuf.at[1], q1_hbm.at[wid])


@functools.cache
def _p1():
    return pl.kernel(
        _p1_real_body,
        out_type=(
            jax.ShapeDtypeStruct((E, C), jnp.float32),
            jax.ShapeDtypeStruct((NW, 1, C), jnp.float32),
            jax.ShapeDtypeStruct((NW, 1, C), jnp.float32),
            jax.ShapeDtypeStruct((NC * NHR, C), jnp.float32),
        ),
        mesh=_mesh(),
        scratch_types=[
            pltpu.VMEM((NCH, CH), jnp.int32),
            pltpu.VMEM((NCH, CH), jnp.int32),
            pltpu.VMEM((2, CH), jnp.int32),
            pltpu.VMEM((2, CH, C), jnp.float32),
            pltpu.VMEM((2, CH, C), jnp.float32),
            pltpu.VMEM((2, CH, C), jnp.float32),
            pltpu.VMEM((2, CH, C), jnp.float32),
            pltpu.VMEM((2, 1, C), jnp.float32),
            pltpu.VMEM_SHARED((NHR, C), jnp.float32),
            pltpu.SemaphoreType.DMA,
            pltpu.SemaphoreType.DMA,
            pltpu.SemaphoreType.DMA,
            pltpu.SemaphoreType.DMA,
            pltpu.SemaphoreType.DMA,
        ],
    )


_P2_RB = 2000
_P2_NSTEP = E // _P2_RB


def _p2_body(s1_ref, q1_ref, g1_ref, be1_ref, w2_ref, g2_ref, be2_ref,
             h1_ref, h2_ref, ab2_ref, acc_ref):
    i = pl.program_id(0)
    s1 = jnp.sum(s1_ref[...], axis=0)
    q1 = jnp.sum(q1_ref[...], axis=0)
    m1 = s1 / E
    var1 = q1 / E - m1 * m1
    inv1 = lax.rsqrt(var1 + EPS)
    a1 = g1_ref[0, :] * inv1
    c1 = be1_ref[0, :] - m1 * a1
    z = jnp.maximum(h1_ref[...] * a1[None, :] + c1[None, :], 0.0)
    h2 = jnp.dot(z, w2_ref[...], preferred_element_type=jnp.float32)
    h2_ref[...] = h2

    @pl.when(i == 0)
    def _():
        acc_ref[...] = jnp.zeros_like(acc_ref)
        ab2_ref[...] = jnp.zeros_like(ab2_ref)

    acc_ref[0, :] += jnp.sum(h2, axis=0)
    acc_ref[1, :] += jnp.sum(h2 * h2, axis=0)

    @pl.when(i == _P2_NSTEP - 1)
    def _():
        m2 = acc_ref[0, :] / E
        var2 = acc_ref[1, :] / E - m2 * m2
        inv2 = lax.rsqrt(var2 + EPS)
        a2 = g2_ref[0, :] * inv2
        c2 = be2_ref[0, :] - m2 * a2
        ab2_ref[0, :] = a2
        ab2_ref[1, :] = c2


def _p2(s1, q1, g1, be1, W2, g2, be2, h1):
    return pl.pallas_call(
        _p2_body,
        grid=(_P2_NSTEP,),
        in_specs=[
            pl.BlockSpec((NW, C), lambda i: (0, 0)),
            pl.BlockSpec((NW, C), lambda i: (0, 0)),
            pl.BlockSpec((1, C), lambda i: (0, 0)),
            pl.BlockSpec((1, C), lambda i: (0, 0)),
            pl.BlockSpec((C, C), lambda i: (0, 0)),
            pl.BlockSpec((1, C), lambda i: (0, 0)),
            pl.BlockSpec((1, C), lambda i: (0, 0)),
            pl.BlockSpec((_P2_RB, C), lambda i: (i, 0)),
        ],
        out_specs=[
            pl.BlockSpec((_P2_RB, C), lambda i: (i, 0)),
            pl.BlockSpec((8, C), lambda i: (0, 0)),
        ],
        out_shape=[
            jax.ShapeDtypeStruct((E, C), jnp.float32),
            jax.ShapeDtypeStruct((8, C), jnp.float32),
        ],
        scratch_shapes=[pltpu.VMEM((8, C), jnp.float32)],
    )(s1, q1, g1, be1, W2, g2, be2, h1)


def _p3_body(h2_hbm, dst_hbm, ab2_hbm, part_hbm,
             idxd, hbuf, zrow, abuf, acc_sh,
             sem_h, sem_s, sem_z):
    cid = lax.axis_index("c")
    sid = lax.axis_index("s")
    wid = sid * NC + cid
    base = wid * EPW
    row0 = sid * NPS
    zero_v = jnp.zeros((NL,), jnp.float32)

    cpi = pltpu.async_copy(dst_hbm.at[wid], idxd, sem_z)
    cpa = pltpu.async_copy(ab2_hbm, abuf, sem_z)

    def fill_zrow(r, _):
        for g in range(CG):
            zrow[r, pl.ds(g * NL, NL)] = zero_v
        return 0

    lax.fori_loop(0, CH, fill_zrow, 0)
    cpi.wait()
    cpa.wait()

    nfull = NPS // CH
    for k in range(nfull):
        pltpu.async_copy(zrow, acc_sh.at[pl.ds(row0 + k * CH, CH)], sem_z)
    for k in range(nfull):
        pltpu.make_async_copy(zrow, acc_sh.at[pl.ds(row0, CH)], sem_z).wait()

    plsc.subcore_barrier()

    a2 = [abuf[0, pl.ds(g * NL, NL)] for g in range(CG)]
    c2 = [abuf[1, pl.ds(g * NL, NL)] for g in range(CG)]

    pltpu.async_copy(h2_hbm.at[pl.ds(base, CH)], hbuf.at[0], sem_h)

    def do_chunk(j, b):
        pltpu.make_async_copy(
            h2_hbm.at[pl.ds(base + j * CH, CH)], hbuf.at[b], sem_h
        ).wait()

        @pl.when(j >= 1)
        def _():
            pltpu.make_async_copy(
                hbuf.at[1 - b], acc_sh.at[idxd.at[j - 1]], sem_s
            ).wait()

        @pl.when(j + 1 < NCH)
        def _():
            pltpu.async_copy(
                h2_hbm.at[pl.ds(base + (j + 1) * CH, CH)], hbuf.at[1 - b], sem_h
            )

        def edge(e, _c):
            for g in range(CG):
                sl = pl.ds(g * NL, NL)
                h = hbuf[b, e, sl]
                hbuf[b, e, sl] = jnp.maximum(h * a2[g] + c2[g], 0.0)
            return 0

        lax.fori_loop(0, CH, edge, 0)
        pltpu.async_copy(hbuf.at[b], acc_sh.at[idxd.at[j]], sem_s, add=True)

    def outer(io, carry):
        do_chunk(io * 2, 0)
        do_chunk(io * 2 + 1, 1)
        return carry

    lax.fori_loop(0, NCH // 2, outer, 0)
    do_chunk(NCH - 1, 0)

    pltpu.make_async_copy(hbuf.at[0], acc_sh.at[idxd.at[NCH - 1]], sem_s).wait()

    plsc.subcore_barrier()

    cpo = pltpu.async_copy(
        acc_sh.at[pl.ds(row0, NPS)], part_hbm.at[pl.ds(cid * NP + row0, NPS)],
        sem_z,
    )
    cpo.wait()


@functools.cache
def _p3():
    return pl.kernel(
        _p3_body,
        out_type=jax.ShapeDtypeStruct((NC * NP, C), jnp.float32),
        mesh=_mesh(),
        scratch_types=[
            pltpu.VMEM((NCH, CH), jnp.int32),
            pltpu.VMEM((2, CH, C), jnp.float32),
            pltpu.VMEM((CH, C), jnp.float32),
            pltpu.VMEM((8, C), jnp.float32),
            pltpu.VMEM_SHARED((NP, C), jnp.float32),
            pltpu.SemaphoreType.DMA,
            pltpu.SemaphoreType.DMA,
            pltpu.SemaphoreType.DMA,
        ],
    )


_P4_RB = 1280


_P4_HR = _P4_RB // 8


def _p4_body(p0_ref, p1_ref, h0_ref, h1_ref, out_ref):
    hs = h0_ref[0] + h1_ref[0]
    col = lax.broadcasted_iota(jnp.int32, (C, 8), 0) // NL
    grp = lax.broadcasted_iota(jnp.int32, (C, 8), 1)
    sel = jnp.where(col == grp, 1.0, 0.0)
    cnt = jnp.dot(hs, sel, preferred_element_type=jnp.float32)
    den = jnp.clip(cnt, 1.0, None)
    p = p0_ref[0] + p1_ref[0]
    out = p.reshape(_P4_HR, 8, C) / den[:, :, None]
    out_ref[...] = out.reshape(_P4_RB, C)


def _p4(part, hist):
    nb = NP // _P4_RB
    part = part.reshape(NC, NP, C)
    hist = hist.reshape(NC, NHR, C)
    return pl.pallas_call(
        _p4_body,
        grid=(nb,),
        in_specs=[
            pl.BlockSpec((1, _P4_RB, C), lambda i: (0, i, 0)),
            pl.BlockSpec((1, _P4_RB, C), lambda i: (1, i, 0)),
            pl.BlockSpec((1, _P4_HR, C), lambda i: (0, i, 0)),
            pl.BlockSpec((1, _P4_HR, C), lambda i: (1, i, 0)),
        ],
        out_specs=pl.BlockSpec((_P4_RB, C), lambda i: (i, 0)),
        out_shape=jax.ShapeDtypeStruct((NP, C), jnp.float32),
    )(part, part, hist, hist)


@jax.jit
def kernel(x, edge_index, W1, b1, g1, be1, W2, b2, g2, be2):
    del b1, b2
    src = edge_index[0].reshape(NW, NCH, CH)
    dst = edge_index[1].reshape(NW, NCH, CH)
    u, v = _p0(x, W1)
    h1, s1, q1, hist = _p1()(u, v, dst, src)
    s1 = s1.reshape(NW, C)
    q1 = q1.reshape(NW, C)
    h2, ab2 = _p2(s1, q1, g1.reshape(1, C), be1.reshape(1, C), W2,
                  g2.reshape(1, C), be2.reshape(1, C), h1)
    part = _p3()(h2, dst, ab2)
    out = _p4(part, hist)
    return out[:N]
